# baseline (speedup 1.0000x reference)
import functools

import jax
import jax.numpy as jnp
from jax import lax
from jax.experimental import pallas as pl
from jax.experimental.pallas import tpu as pltpu
from jax.experimental.pallas import tpu_sc as plsc

_CODEBOOK_NUM = 2
_CENTROIDS = 256
_CENTROID_LEN = 4
_ROWS = 4096
_COLS = 4096
_PER_BOOK = _ROWS * _COLS // _CENTROID_LEN // _CODEBOOK_NUM

_NC = 2
_NS = 16
_NW = _NC * _NS

_ROWS_PER_W = _ROWS // _NW
_CODES_PER_ROW = _COLS // _CENTROID_LEN
_CHUNK_ROWS = 8
_CHUNKS = _ROWS_PER_W // _CHUNK_ROWS
_CODES_PER_CHUNK = _CHUNK_ROWS * _CODES_PER_ROW
_GROUPS_PER_ROW = _CODES_PER_ROW // 16
_UNROLL = 2


def _dequant_body(codes_hbm, books_hbm, scales_hbm, out_hbm,
                  table_v, book_v, scales_v, codes_v0, codes_v1, out_v0, out_v1,
                  in_sem0, in_sem1, out_sem0, out_sem1):
    codes_bufs = [codes_v0, codes_v1]
    out_bufs = [out_v0, out_v1]
    wid = lax.axis_index("s") * _NC + lax.axis_index("c")
    row0 = wid * _ROWS_PER_W
    book = wid // (_NW // 2)
    boff = (wid % (_NW // 2)) * (_ROWS_PER_W * _CODES_PER_ROW)

    pltpu.sync_copy(books_hbm.at[pl.ds(book, 1), :], book_v)
    pltpu.sync_copy(scales_hbm.at[pl.ds(row0, _ROWS_PER_W)], scales_v)

    iota = lax.iota(jnp.int32, 16)
    rep_patterns = [(iota >> 2) + 4 * j for j in range(4)]
    zero16 = jnp.full((16,), 0, jnp.int32)
    elem_pat = iota & 3

    @plsc.parallel_loop(0, _CENTROIDS, unroll=4)
    def expand_body(m):
        vals = plsc.load_gather(book_v, [zero16, elem_pat + m * 4])
        table_v[pl.ds(m * 16, 16)] = vals

    in_sems = [in_sem0, in_sem1]
    out_sems = [out_sem0, out_sem1]

    def in_desc(c, buf):
        return pltpu.make_async_copy(
            codes_hbm.at[pl.ds(book, 1),
                         pl.ds(boff + c * _CODES_PER_CHUNK, _CODES_PER_CHUNK)],
            codes_bufs[buf], in_sems[buf])

    def out_desc(c, buf):
        return pltpu.make_async_copy(
            out_bufs[buf],
            out_hbm.at[pl.ds(row0 + c * _CHUNK_ROWS, _CHUNK_ROWS), :],
            out_sems[buf])

    def compute(buf, c):
        codes_ref = codes_bufs[buf]
        out_ref = out_bufs[buf]

        def row_body(r, _):
            row = c * _CHUNK_ROWS + r
            scale = plsc.load_gather(scales_v, [zero16 + row])

            @plsc.parallel_loop(0, _GROUPS_PER_ROW, unroll=_UNROLL)
            def group_body(g):
                code_v = codes_ref[0, pl.ds(r * _CODES_PER_ROW + g * 16, 16)]
                for j in range(4):
                    rep = jnp.take_along_axis(code_v, rep_patterns[j], axis=0)
                    idx = (rep << 4) | iota
                    gj = plsc.load_gather(table_v, [idx])
                    out_ref[r, pl.ds(g * 64 + 16 * j, 16)] = gj * scale

            return 0

        lax.fori_loop(0, _CHUNK_ROWS, row_body, 0)

    in_desc(0, 0).start()
    in_desc(1, 1).start()

    def pair_body(s, _):
        for b in range(2):
            c = s * 2 + b
            in_desc(c, b).wait()

            @pl.when(s > 0)
            def _():
                out_desc(c, b).wait()

            compute(b, c)

            @pl.when(s < _CHUNKS // 2 - 1)
            def _():
                in_desc(c + 2, b).start()

            out_desc(c, b).start()
        return 0

    lax.fori_loop(0, _CHUNKS // 2, pair_body, 0)

    out_desc(_CHUNKS - 2, 0).wait()
    out_desc(_CHUNKS - 1, 1).wait()


@jax.jit
def _dequant(codes, codebooks, scales):
    mesh = plsc.VectorSubcoreMesh(
        core_axis_name="c", subcore_axis_name="s",
        num_cores=_NC, num_subcores=_NS)
    kfn = pl.kernel(
        _dequant_body,
        out_type=jax.ShapeDtypeStruct((_ROWS, _COLS), jnp.float32),
        mesh=mesh,
        compiler_params=pltpu.CompilerParams(needs_layout_passes=False),
        scratch_types=[
            pltpu.VMEM((_CENTROIDS * 16,), jnp.float32),
            pltpu.VMEM((1, _CENTROIDS * _CENTROID_LEN), jnp.float32),
            pltpu.VMEM((_ROWS_PER_W,), jnp.float32),
            pltpu.VMEM((1, _CODES_PER_CHUNK), jnp.int32),
            pltpu.VMEM((1, _CODES_PER_CHUNK), jnp.int32),
            pltpu.VMEM((_CHUNK_ROWS, _COLS), jnp.float32),
            pltpu.VMEM((_CHUNK_ROWS, _COLS), jnp.float32),
            pltpu.SemaphoreType.DMA,
            pltpu.SemaphoreType.DMA,
            pltpu.SemaphoreType.DMA,
            pltpu.SemaphoreType.DMA,
        ],
    )
    return kfn(codes, codebooks, scales)


def kernel(codes, codebooks, scales):
    return _dequant(codes, codebooks.reshape(_CODEBOOK_NUM, -1),
                    scales.reshape(-1))


# speedup vs baseline: 1.0635x; 1.0635x over previous
import functools

import jax
import jax.numpy as jnp
from jax import lax
from jax.experimental import pallas as pl
from jax.experimental.pallas import tpu as pltpu
from jax.experimental.pallas import tpu_sc as plsc

_CODEBOOK_NUM = 2
_CENTROIDS = 256
_CENTROID_LEN = 4
_ROWS = 4096
_COLS = 4096
_PER_BOOK = _ROWS * _COLS // _CENTROID_LEN // _CODEBOOK_NUM

_NC = 2
_NS = 16
_NW = _NC * _NS

_ROWS_PER_W = _ROWS // _NW
_CODES_PER_ROW = _COLS // _CENTROID_LEN
_CHUNK_ROWS = 8
_CHUNKS = _ROWS_PER_W // _CHUNK_ROWS
_CODES_PER_CHUNK = _CHUNK_ROWS * _CODES_PER_ROW
_GROUPS_PER_ROW = _CODES_PER_ROW // 16
_UNROLL = 4


def _dequant_body(codes_hbm, books_hbm, scales_hbm, out_hbm,
                  table_v, book_v, scales_v, codes_v0, codes_v1, out_v0, out_v1,
                  in_sem0, in_sem1, out_sem0, out_sem1):
    codes_bufs = [codes_v0, codes_v1]
    out_bufs = [out_v0, out_v1]
    wid = lax.axis_index("s") * _NC + lax.axis_index("c")
    row0 = wid * _ROWS_PER_W
    book = wid // (_NW // 2)
    boff = (wid % (_NW // 2)) * (_ROWS_PER_W * _CODES_PER_ROW)

    iota = lax.iota(jnp.int32, 16)
    rep_patterns = [(iota >> 2) + 4 * j for j in range(4)]
    zero16 = jnp.full((16,), 0, jnp.int32)
    elem_pat = iota & 3

    in_sems = [in_sem0, in_sem1]
    out_sems = [out_sem0, out_sem1]

    def in_desc(c, buf):
        return pltpu.make_async_copy(
            codes_hbm.at[pl.ds(book, 1),
                         pl.ds(boff + c * _CODES_PER_CHUNK, _CODES_PER_CHUNK)],
            codes_bufs[buf], in_sems[buf])

    def out_desc(c, buf):
        return pltpu.make_async_copy(
            out_bufs[buf],
            out_hbm.at[pl.ds(row0 + c * _CHUNK_ROWS, _CHUNK_ROWS), :],
            out_sems[buf])

    def compute(buf, c):
        codes_ref = codes_bufs[buf]
        out_ref = out_bufs[buf]

        def row_body(r, _):
            row = c * _CHUNK_ROWS + r
            scale = plsc.load_gather(scales_v, [zero16 + row])

            @plsc.parallel_loop(0, _GROUPS_PER_ROW, unroll=_UNROLL)
            def group_body(g):
                code_v = codes_ref[0, pl.ds(r * _CODES_PER_ROW + g * 16, 16)]
                for j in range(4):
                    rep = jnp.take_along_axis(code_v, rep_patterns[j], axis=0)
                    idx = (rep << 4) | iota
                    gj = plsc.load_gather(table_v, [idx])
                    out_ref[r, pl.ds(g * 64 + 16 * j, 16)] = gj * scale

            return 0

        lax.fori_loop(0, _CHUNK_ROWS, row_body, 0)

    in_desc(0, 0).start()
    in_desc(1, 1).start()

    pltpu.sync_copy(books_hbm.at[pl.ds(book, 1), :], book_v)
    pltpu.sync_copy(scales_hbm.at[pl.ds(row0, _ROWS_PER_W)], scales_v)

    @plsc.parallel_loop(0, _CENTROIDS, unroll=4)
    def expand_body(m):
        vals = plsc.load_gather(book_v, [zero16, elem_pat + m * 4])
        table_v[pl.ds(m * 16, 16)] = vals

    def pair_body(s, _):
        for b in range(2):
            c = s * 2 + b
            in_desc(c, b).wait()

            @pl.when(s > 0)
            def _():
                out_desc(c, b).wait()

            compute(b, c)

            @pl.when(s < _CHUNKS // 2 - 1)
            def _():
                in_desc(c + 2, b).start()

            out_desc(c, b).start()
        return 0

    lax.fori_loop(0, _CHUNKS // 2, pair_body, 0)

    out_desc(_CHUNKS - 2, 0).wait()
    out_desc(_CHUNKS - 1, 1).wait()


@jax.jit
def _dequant(codes, codebooks, scales):
    mesh = plsc.VectorSubcoreMesh(
        core_axis_name="c", subcore_axis_name="s",
        num_cores=_NC, num_subcores=_NS)
    kfn = pl.kernel(
        _dequant_body,
        out_type=jax.ShapeDtypeStruct((_ROWS, _COLS), jnp.float32),
        mesh=mesh,
        compiler_params=pltpu.CompilerParams(needs_layout_passes=False),
        scratch_types=[
            pltpu.VMEM((_CENTROIDS * 16,), jnp.float32),
            pltpu.VMEM((1, _CENTROIDS * _CENTROID_LEN), jnp.float32),
            pltpu.VMEM((_ROWS_PER_W,), jnp.float32),
            pltpu.VMEM((1, _CODES_PER_CHUNK), jnp.int32),
            pltpu.VMEM((1, _CODES_PER_CHUNK), jnp.int32),
            pltpu.VMEM((_CHUNK_ROWS, _COLS), jnp.float32),
            pltpu.VMEM((_CHUNK_ROWS, _COLS), jnp.float32),
            pltpu.SemaphoreType.DMA,
            pltpu.SemaphoreType.DMA,
            pltpu.SemaphoreType.DMA,
            pltpu.SemaphoreType.DMA,
        ],
    )
    return kfn(codes, codebooks, scales)


def kernel(codes, codebooks, scales):
    return _dequant(codes, codebooks.reshape(_CODEBOOK_NUM, -1),
                    scales.reshape(-1))


# speedup vs baseline: 1.0672x; 1.0034x over previous
import jax
import jax.numpy as jnp
from jax import lax
from jax.experimental import pallas as pl
from jax.experimental.pallas import tpu as pltpu
from jax.experimental.pallas import tpu_sc as plsc

_CODEBOOK_NUM = 2
_CENTROIDS = 256
_CENTROID_LEN = 4
_ROWS = 4096
_COLS = 4096

_NC = 2
_NS = 16
_NW = _NC * _NS

_ROWS_PER_W = _ROWS // _NW
_CODES_PER_ROW = _COLS // _CENTROID_LEN
_CHUNK_ROWS = 8
_CHUNKS = _ROWS_PER_W // _CHUNK_ROWS
_CODES_PER_CHUNK = _CHUNK_ROWS * _CODES_PER_ROW
_GROUPS_PER_ROW = _CODES_PER_ROW // 16
_UNROLL = 4


def _dequant_body(codes_hbm, books_hbm, scales_hbm, out_hbm,
                  table_v, book_v, scales_v, codes_v0, codes_v1, out_v0, out_v1,
                  in_sem0, in_sem1, out_sem0, out_sem1):
    codes_bufs = [codes_v0, codes_v1]
    out_bufs = [out_v0, out_v1]
    wid = lax.axis_index("s") * _NC + lax.axis_index("c")
    row0 = wid * _ROWS_PER_W
    book = wid // (_NW // 2)
    boff = (wid % (_NW // 2)) * (_ROWS_PER_W * _CODES_PER_ROW)

    iota = lax.iota(jnp.int32, 16)
    rep_patterns = [(iota >> 2) + 4 * j for j in range(4)]
    zero16 = jnp.full((16,), 0, jnp.int32)
    elem_pat = iota & 3

    in_sems = [in_sem0, in_sem1]
    out_sems = [out_sem0, out_sem1]

    def in_desc(c, buf):
        return pltpu.make_async_copy(
            codes_hbm.at[pl.ds(book, 1),
                         pl.ds(boff + c * _CODES_PER_CHUNK, _CODES_PER_CHUNK)],
            codes_bufs[buf], in_sems[buf])

    def out_desc(c, buf):
        return pltpu.make_async_copy(
            out_bufs[buf],
            out_hbm.at[pl.ds(row0 + c * _CHUNK_ROWS, _CHUNK_ROWS), :],
            out_sems[buf])

    def compute(buf, c):
        codes_ref = codes_bufs[buf]
        out_ref = out_bufs[buf]

        def row_body(r, _):
            row = c * _CHUNK_ROWS + r
            scale = plsc.load_gather(scales_v, [zero16 + row])

            @plsc.parallel_loop(0, _GROUPS_PER_ROW, unroll=_UNROLL)
            def group_body(g):
                code_v = codes_ref[0, pl.ds(r * _CODES_PER_ROW + g * 16, 16)]
                for j in range(4):
                    rep = jnp.take_along_axis(code_v, rep_patterns[j], axis=0)
                    idx = (rep << 4) | iota
                    gj = plsc.load_gather(table_v, [idx])
                    out_ref[r, pl.ds(g * 64 + 16 * j, 16)] = gj * scale

            return 0

        lax.fori_loop(0, _CHUNK_ROWS, row_body, 0)

    in_desc(0, 0).start()
    in_desc(1, 1).start()

    pltpu.sync_copy(books_hbm.at[pl.ds(book, 1), :], book_v)
    pltpu.sync_copy(scales_hbm.at[pl.ds(row0, _ROWS_PER_W)], scales_v)

    @plsc.parallel_loop(0, _CENTROIDS, unroll=4)
    def expand_body(m):
        vals = plsc.load_gather(book_v, [zero16, elem_pat + m * 4])
        table_v[pl.ds(m * 16, 16)] = vals

    def pair_body(s, _):
        for b in range(2):
            c = s * 2 + b
            in_desc(c, b).wait()

            @pl.when(s > 0)
            def _():
                out_desc(c, b).wait()

            compute(b, c)

            @pl.when(s < _CHUNKS // 2 - 1)
            def _():
                in_desc(c + 2, b).start()

            out_desc(c, b).start()
        return 0

    lax.fori_loop(0, _CHUNKS // 2, pair_body, 0)

    out_desc(_CHUNKS - 2, 0).wait()
    out_desc(_CHUNKS - 1, 1).wait()


@jax.jit
def _dequant(codes, codebooks, scales):
    mesh = plsc.VectorSubcoreMesh(
        core_axis_name="c", subcore_axis_name="s",
        num_cores=_NC, num_subcores=_NS)
    kfn = pl.kernel(
        _dequant_body,
        out_type=jax.ShapeDtypeStruct((_ROWS, _COLS), jnp.float32),
        mesh=mesh,
        compiler_params=pltpu.CompilerParams(needs_layout_passes=False),
        scratch_types=[
            pltpu.VMEM((_CENTROIDS * 16,), jnp.float32),
            pltpu.VMEM((1, _CENTROIDS * _CENTROID_LEN), jnp.float32),
            pltpu.VMEM((_ROWS_PER_W,), jnp.float32),
            pltpu.VMEM((1, _CODES_PER_CHUNK), jnp.int32),
            pltpu.VMEM((1, _CODES_PER_CHUNK), jnp.int32),
            pltpu.VMEM((_CHUNK_ROWS, _COLS), jnp.float32),
            pltpu.VMEM((_CHUNK_ROWS, _COLS), jnp.float32),
            pltpu.SemaphoreType.DMA,
            pltpu.SemaphoreType.DMA,
            pltpu.SemaphoreType.DMA,
            pltpu.SemaphoreType.DMA,
        ],
    )
    return kfn(codes, codebooks, scales)


def kernel(codes, codebooks, scales):
    return _dequant(codes, codebooks.reshape(_CODEBOOK_NUM, -1),
                    scales.reshape(-1))
